# baseline (speedup 1.0000x reference)
import math

import jax
import jax.numpy as jnp
from jax.experimental import pallas as pl
from jax.experimental.pallas import tpu as pltpu

B, L, PATCH_DIM, H = 8, 1024, 768, 1024
FREQ = 256
HALF = FREQ // 2
_LOG_MAX_PERIOD = math.log(10000.0)


def _body(fs_ref, px_ref, w_ref, pos_ref, cls_ref, w0_ref, b0_ref,
          w2_ref, b2_ref, out_ref):
    b = pl.program_id(0)
    px = px_ref[0].astype(jnp.bfloat16)
    x = jax.lax.dot_general(
        px, w_ref[...], (((1,), (1,)), ((), ())),
        preferred_element_type=jnp.float32)
    out_ref[0, 2:, :] = x + pos_ref[...]
    k = jax.lax.broadcasted_iota(jnp.int32, (1, HALF), 1).astype(jnp.float32)
    freqs = jnp.exp((-_LOG_MAX_PERIOD / HALF) * k)
    args = fs_ref[...] * freqs
    emb = jnp.concatenate([jnp.cos(args), jnp.sin(args)], axis=-1)
    t = jax.lax.dot_general(
        emb, w0_ref[...], (((1,), (1,)), ((), ())),
        preferred_element_type=jnp.float32) + b0_ref[...]
    t = t * jax.nn.sigmoid(t)
    tok = jax.lax.dot_general(
        t, w2_ref[...], (((1,), (1,)), ((), ())),
        preferred_element_type=jnp.float32) + b2_ref[...]
    rowmask = jax.lax.broadcasted_iota(jnp.int32, (B, 1), 0) == b
    out_ref[0, pl.ds(1, 1), :] = jnp.sum(
        jnp.where(rowmask, tok, 0.0), axis=0, keepdims=True)
    out_ref[0, pl.ds(0, 1), :] = cls_ref[...]


def kernel(pixel_values, fs, proj_w, proj_b, pos_emb, cls_token,
           fs_w0, fs_b0, fs_w2, fs_b2):
    w_bf = proj_w.astype(jnp.bfloat16)
    pos_pb = pos_emb[:L] + proj_b[None, :]
    cls2 = cls_token.reshape(1, H)
    fs2 = fs.reshape(B, 1)
    b0 = fs_b0.reshape(1, H)
    b2 = fs_b2.reshape(1, H)

    const = lambda *_: (0, 0)
    out = pl.pallas_call(
        _body,
        grid=(B,),
        in_specs=[
            pl.BlockSpec((B, 1), const),
            pl.BlockSpec((1, L, PATCH_DIM), lambda b: (b, 0, 0)),
            pl.BlockSpec((H, PATCH_DIM), const),
            pl.BlockSpec((L, H), const),
            pl.BlockSpec((1, H), const),
            pl.BlockSpec((H, FREQ), const),
            pl.BlockSpec((1, H), const),
            pl.BlockSpec((H, H), const),
            pl.BlockSpec((1, H), const),
        ],
        out_specs=pl.BlockSpec((1, L + 2, H), lambda b: (b, 0, 0)),
        out_shape=jax.ShapeDtypeStruct((B, L + 2, H), jnp.float32),
        compiler_params=pltpu.CompilerParams(
            dimension_semantics=("parallel",)),
    )(fs2, pixel_values, w_bf, pos_pb, cls2, fs_w0, b0, fs_w2, b2)
    return out


# speedup vs baseline: 1.0253x; 1.0253x over previous
import math

import jax
import jax.numpy as jnp
from jax.experimental import pallas as pl
from jax.experimental.pallas import tpu as pltpu

B, L, PATCH_DIM, H = 8, 1024, 768, 1024
FREQ = 256
HALF = FREQ // 2
_LOG_MAX_PERIOD = math.log(10000.0)


def _body(fs_ref, px_ref, w_ref, pos_ref, cls_ref, w0_ref, b0_ref,
          w2_ref, b2_ref, out_ref, tok_ref):
    b = pl.program_id(0)

    @pl.when(b == 0)
    def _fs_tokens():
        k = jax.lax.broadcasted_iota(jnp.int32, (1, HALF), 1).astype(jnp.float32)
        freqs = jnp.exp((-_LOG_MAX_PERIOD / HALF) * k)
        args = fs_ref[...] * freqs
        emb = jnp.concatenate([jnp.cos(args), jnp.sin(args)], axis=-1)
        t = jax.lax.dot_general(
            emb, w0_ref[...], (((1,), (1,)), ((), ())),
            preferred_element_type=jnp.float32) + b0_ref[...]
        t = t * jax.nn.sigmoid(t)
        tok_ref[...] = jax.lax.dot_general(
            t, w2_ref[...], (((1,), (1,)), ((), ())),
            preferred_element_type=jnp.float32) + b2_ref[...]

    px = px_ref[0].astype(jnp.bfloat16)
    x = jax.lax.dot_general(
        px, w_ref[...], (((1,), (1,)), ((), ())),
        preferred_element_type=jnp.float32)
    out_ref[0, 2:, :] = x + pos_ref[...]
    out_ref[0, pl.ds(1, 1), :] = tok_ref[pl.ds(b, 1), :]
    out_ref[0, pl.ds(0, 1), :] = cls_ref[...]


def kernel(pixel_values, fs, proj_w, proj_b, pos_emb, cls_token,
           fs_w0, fs_b0, fs_w2, fs_b2):
    w_bf = proj_w.astype(jnp.bfloat16)
    pos_pb = pos_emb[:L] + proj_b[None, :]
    cls2 = cls_token.reshape(1, H)
    fs2 = fs.reshape(B, 1)
    b0 = fs_b0.reshape(1, H)
    b2 = fs_b2.reshape(1, H)

    const = lambda *_: (0, 0)
    out = pl.pallas_call(
        _body,
        grid=(B,),
        in_specs=[
            pl.BlockSpec((B, 1), const),
            pl.BlockSpec((1, L, PATCH_DIM), lambda b: (b, 0, 0)),
            pl.BlockSpec((H, PATCH_DIM), const),
            pl.BlockSpec((L, H), const),
            pl.BlockSpec((1, H), const),
            pl.BlockSpec((H, FREQ), const),
            pl.BlockSpec((1, H), const),
            pl.BlockSpec((H, H), const),
            pl.BlockSpec((1, H), const),
        ],
        out_specs=pl.BlockSpec((1, L + 2, H), lambda b: (b, 0, 0)),
        out_shape=jax.ShapeDtypeStruct((B, L + 2, H), jnp.float32),
        scratch_shapes=[pltpu.VMEM((B, H), jnp.float32)],
        compiler_params=pltpu.CompilerParams(
            dimension_semantics=("arbitrary",)),
    )(fs2, pixel_values, w_bf, pos_pb, cls2, fs_w0, b0, fs_w2, b2)
    return out


# speedup vs baseline: 1.1272x; 1.0994x over previous
import math

import jax
import jax.numpy as jnp
from jax.experimental import pallas as pl
from jax.experimental.pallas import tpu as pltpu

B, L, PATCH_DIM, H = 8, 1024, 768, 1024
FREQ = 256
HALF = FREQ // 2
_LOG_MAX_PERIOD = math.log(10000.0)


def _body(fs_ref, px_ref, w_ref, pos_ref, pb_ref, cls_ref, w0_ref, b0_ref,
          w2_ref, b2_ref, out_ref, tok_ref):
    b = pl.program_id(0)

    @pl.when(b == 0)
    def _fs_tokens():
        k = jax.lax.broadcasted_iota(jnp.int32, (1, HALF), 1).astype(jnp.float32)
        freqs = jnp.exp((-_LOG_MAX_PERIOD / HALF) * k)
        args = fs_ref[...] * freqs
        emb = jnp.concatenate([jnp.cos(args), jnp.sin(args)], axis=-1)
        t = jax.lax.dot_general(
            emb, w0_ref[...], (((1,), (1,)), ((), ())),
            preferred_element_type=jnp.float32) + b0_ref[...]
        t = t * jax.nn.sigmoid(t)
        tok_ref[...] = jax.lax.dot_general(
            t, w2_ref[...], (((1,), (1,)), ((), ())),
            preferred_element_type=jnp.float32) + b2_ref[...]

    px = px_ref[0].astype(jnp.bfloat16)
    x = jax.lax.dot_general(
        px, w_ref[...].astype(jnp.bfloat16), (((1,), (1,)), ((), ())),
        preferred_element_type=jnp.float32)
    out_ref[0, 2:, :] = x + pos_ref[...] + pb_ref[...]
    out_ref[0, pl.ds(1, 1), :] = tok_ref[pl.ds(b, 1), :]
    out_ref[0, pl.ds(0, 1), :] = cls_ref[...]


def kernel(pixel_values, fs, proj_w, proj_b, pos_emb, cls_token,
           fs_w0, fs_b0, fs_w2, fs_b2):
    cls2 = cls_token.reshape(1, H)
    fs2 = fs.reshape(B, 1)
    pb = proj_b.reshape(1, H)
    b0 = fs_b0.reshape(1, H)
    b2 = fs_b2.reshape(1, H)

    const = lambda *_: (0, 0)
    out = pl.pallas_call(
        _body,
        grid=(B,),
        in_specs=[
            pl.BlockSpec((B, 1), const),
            pl.BlockSpec((1, L, PATCH_DIM), lambda b: (b, 0, 0)),
            pl.BlockSpec((H, PATCH_DIM), const),
            pl.BlockSpec((L, H), const),
            pl.BlockSpec((1, H), const),
            pl.BlockSpec((1, H), const),
            pl.BlockSpec((H, FREQ), const),
            pl.BlockSpec((1, H), const),
            pl.BlockSpec((H, H), const),
            pl.BlockSpec((1, H), const),
        ],
        out_specs=pl.BlockSpec((1, L + 2, H), lambda b: (b, 0, 0)),
        out_shape=jax.ShapeDtypeStruct((B, L + 2, H), jnp.float32),
        scratch_shapes=[pltpu.VMEM((B, H), jnp.float32)],
        compiler_params=pltpu.CompilerParams(
            dimension_semantics=("arbitrary",)),
    )(fs2, pixel_values, proj_w, pos_emb, pb, cls2, fs_w0, b0, fs_w2, b2)
    return out


# speedup vs baseline: 2.2187x; 1.9683x over previous
import math

import jax
import jax.numpy as jnp
from jax.experimental import pallas as pl
from jax.experimental.pallas import tpu as pltpu

B, L, PATCH_DIM, H = 8, 1024, 768, 1024
FREQ = 256
HALF = FREQ // 2
_LOG_MAX_PERIOD = math.log(10000.0)
TB = 128
NT = (L + 2 + TB - 1) // TB


def _body(fs_ref, px_ref, w_ref, pos_ref, pb_ref, cls_ref, w0_ref, b0_ref,
          w2_ref, b2_ref, out_ref, tok_ref, carry_ref, pcarry_ref):
    t = pl.program_id(0)

    cur = jnp.transpose(px_ref[...].astype(jnp.bfloat16), (1, 0, 2))
    asm = jnp.concatenate([carry_ref[...], cur[: TB - 2]], axis=0)
    carry_ref[...] = cur[TB - 2:]
    pos_asm = jnp.concatenate([pcarry_ref[...], pos_ref[: TB - 2]], axis=0)
    pcarry_ref[...] = pos_ref[TB - 2:]

    x = jax.lax.dot_general(
        asm.reshape(TB * B, PATCH_DIM), w_ref[...].astype(jnp.bfloat16),
        (((1,), (1,)), ((), ())),
        preferred_element_type=jnp.float32).reshape(TB, B, H)
    out_ref[...] = x + pos_asm[:, None, :] + pb_ref[...][None]

    @pl.when(t == 0)
    def _head_rows():
        k = jax.lax.broadcasted_iota(jnp.int32, (1, HALF), 1).astype(jnp.float32)
        freqs = jnp.exp((-_LOG_MAX_PERIOD / HALF) * k)
        args = fs_ref[...] * freqs
        emb = jnp.concatenate([jnp.cos(args), jnp.sin(args)], axis=-1)
        tt = jax.lax.dot_general(
            emb, w0_ref[...], (((1,), (1,)), ((), ())),
            preferred_element_type=jnp.float32) + b0_ref[...]
        tt = tt * jax.nn.sigmoid(tt)
        tok_ref[...] = jax.lax.dot_general(
            tt, w2_ref[...], (((1,), (1,)), ((), ())),
            preferred_element_type=jnp.float32) + b2_ref[...]
        out_ref[0, :, :] = jnp.broadcast_to(cls_ref[...], (B, H))
        out_ref[1, :, :] = tok_ref[...]


def kernel(pixel_values, fs, proj_w, proj_b, pos_emb, cls_token,
           fs_w0, fs_b0, fs_w2, fs_b2):
    cls2 = cls_token.reshape(1, H)
    fs2 = fs.reshape(B, 1)
    pb = proj_b.reshape(1, H)
    b0 = fs_b0.reshape(1, H)
    b2 = fs_b2.reshape(1, H)

    const = lambda *_: (0, 0)
    out_t = pl.pallas_call(
        _body,
        grid=(NT,),
        in_specs=[
            pl.BlockSpec((B, 1), const),
            pl.BlockSpec((B, TB, PATCH_DIM),
                         lambda t: (0, jnp.minimum(t, L // TB - 1), 0)),
            pl.BlockSpec((H, PATCH_DIM), const),
            pl.BlockSpec((TB, H),
                         lambda t: (jnp.minimum(t, L // TB - 1), 0)),
            pl.BlockSpec((1, H), const),
            pl.BlockSpec((1, H), const),
            pl.BlockSpec((H, FREQ), const),
            pl.BlockSpec((1, H), const),
            pl.BlockSpec((H, H), const),
            pl.BlockSpec((1, H), const),
        ],
        out_specs=pl.BlockSpec((TB, B, H), lambda t: (t, 0, 0)),
        out_shape=jax.ShapeDtypeStruct((L + 2, B, H), jnp.float32),
        scratch_shapes=[
            pltpu.VMEM((B, H), jnp.float32),
            pltpu.VMEM((2, B, PATCH_DIM), jnp.bfloat16),
            pltpu.VMEM((2, H), jnp.float32),
        ],
        compiler_params=pltpu.CompilerParams(
            dimension_semantics=("arbitrary",)),
    )(fs2, pixel_values, proj_w, pos_emb, pb, cls2, fs_w0, b0, fs_w2, b2)
    return jnp.transpose(out_t, (1, 0, 2))
